# Initial kernel scaffold; baseline (speedup 1.0000x reference)
import jax
import jax.numpy as jnp
from jax.experimental import pallas as pl


def kernel(x, adj, W_att, a_att, g0, b0, g1, b1, g2, b2, g3, b3, Wg, bg, Wfc, bfc):
    raise NotImplementedError("write your pallas kernel here")


# speedup vs baseline: 15.3258x; 15.3258x over previous
import functools

import jax
import jax.numpy as jnp
from jax import lax
from jax.experimental import pallas as pl
from jax.experimental.pallas import tpu as pltpu
from jax.experimental.pallas import tpu_sc as plsc

_N = 10000
_E = 320000
_F = 128
_NH = 4
_NCLS = 40
_EPS = 1e-5
_SM = 0.5

_NP = 10240
_NCORE = 2
_NSUB = 16
_NW = _NCORE * _NSUB
_CHK = 128
_CPT = 79
_EP = _NW * _CPT * _CHK
_RPT = _NP // _NSUB
_DW = 16

_BR = 256
_GRID = _NP // _BR


@functools.cache
def _make_spmm():
    mesh = plsc.VectorSubcoreMesh(
        core_axis_name="c", subcore_axis_name="s",
        num_cores=_NCORE, num_subcores=_NSUB)

    @functools.partial(
        pl.kernel,
        out_type=jax.ShapeDtypeStruct((_NCORE, _NP, _F), jnp.float32),
        mesh=mesh,
        scratch_types=[
            pltpu.VMEM((_CHK,), jnp.int32),
            pltpu.VMEM((_CHK,), jnp.int32),
            pltpu.VMEM((_CHK, _F), jnp.float32),
            pltpu.VMEM_SHARED((_NP, _F), jnp.float32),
            pltpu.SemaphoreType.DMA,
        ],
    )
    def spmm(x_hbm, src_hbm, dst_hbm, zeros_hbm, out_hbm,
             sidx, didx, rows, acc, sem):
        c = lax.axis_index("c")
        s = lax.axis_index("s")
        wid = s * _NCORE + c
        r0 = s * _RPT
        pltpu.sync_copy(zeros_hbm.at[pl.ds(r0, _RPT)],
                        acc.at[pl.ds(r0, _RPT)])
        plsc.subcore_barrier()

        def body(k, carry):
            pltpu.sync_copy(src_hbm.at[wid, k], sidx)
            pltpu.sync_copy(dst_hbm.at[wid, k], didx)
            pltpu.async_copy(x_hbm.at[sidx], rows, sem).wait()
            pltpu.sync_copy(rows, acc.at[didx], add=True)
            return carry

        lax.fori_loop(0, _CPT, body, 0)
        plsc.subcore_barrier()
        pltpu.sync_copy(acc.at[pl.ds(r0, _RPT)],
                        out_hbm.at[c, pl.ds(r0, _RPT)])

    return spmm


@functools.cache
def _make_deg():
    mesh = plsc.VectorSubcoreMesh(
        core_axis_name="c", subcore_axis_name="s",
        num_cores=_NCORE, num_subcores=_NSUB)

    @functools.partial(
        pl.kernel,
        out_type=jax.ShapeDtypeStruct((_NCORE, _NP, _F), jnp.float32),
        mesh=mesh,
        scratch_types=[
            pltpu.VMEM((_CHK,), jnp.int32),
            pltpu.VMEM((_CHK, _F), jnp.float32),
            pltpu.VMEM_SHARED((_NP, _F), jnp.float32),
        ],
    )
    def deg(dst_hbm, ones_hbm, zeros_hbm, out_hbm, didx, ones_v, acc):
        c = lax.axis_index("c")
        s = lax.axis_index("s")
        wid = s * _NCORE + c
        r0 = s * _RPT
        pltpu.sync_copy(zeros_hbm.at[pl.ds(r0, _RPT)],
                        acc.at[pl.ds(r0, _RPT)])
        pltpu.sync_copy(ones_hbm, ones_v)
        plsc.subcore_barrier()

        def body(k, carry):
            pltpu.sync_copy(dst_hbm.at[wid, k], didx)
            pltpu.sync_copy(ones_v, acc.at[didx], add=True)
            return carry

        lax.fori_loop(0, _CPT, body, 0)
        plsc.subcore_barrier()
        pltpu.sync_copy(acc.at[pl.ds(r0, _RPT)],
                        out_hbm.at[c, pl.ds(r0, _RPT)])

    return deg


def _spmm(x_p, src3, dst3, zerosF):
    return _make_spmm()(x_p, src3, dst3, zerosF)


def _deg(dst3, onesD, zerosD):
    return _make_deg()(dst3, onesD, zerosD)


def _row(i):
    return (i, 0)


def _fix(i):
    return (0, 0)


def _prow(i):
    return (0, i, 0)


def _prep_body(degp_ref, x_ref, c0_ref, b0_ref,
               xt_ref, t0_ref, inv_ref, s_ref):
    i = pl.program_id(0)
    deg = degp_ref[0, :, :_DW] + degp_ref[1, :, :_DW]
    deg = jnp.maximum(deg, 1.0)
    rows = lax.broadcasted_iota(jnp.int32, (_BR, _DW), 0) + i * _BR
    valid = rows < _N
    inv = jnp.where(valid, 1.0 / deg, 0.0)
    sq = jnp.where(valid, lax.rsqrt(deg), 0.0)
    inv_ref[...] = inv
    s_ref[...] = sq
    xt = x_ref[...] * c0_ref[...] + b0_ref[...]
    xt_ref[...] = xt
    t0_ref[...] = xt * inv[:, :1]


def _prep(degp, x_p, c0, b0):
    return pl.pallas_call(
        _prep_body,
        grid=(_GRID,),
        in_specs=[
            pl.BlockSpec((_NCORE, _BR, _F), _prow),
            pl.BlockSpec((_BR, _F), _row),
            pl.BlockSpec((1, _F), _fix),
            pl.BlockSpec((1, _F), _fix),
        ],
        out_specs=[
            pl.BlockSpec((_BR, _F), _row),
            pl.BlockSpec((_BR, _F), _row),
            pl.BlockSpec((_BR, _DW), _row),
            pl.BlockSpec((_BR, _DW), _row),
        ],
        out_shape=[
            jax.ShapeDtypeStruct((_NP, _F), jnp.float32),
            jax.ShapeDtypeStruct((_NP, _F), jnp.float32),
            jax.ShapeDtypeStruct((_NP, _DW), jnp.float32),
            jax.ShapeDtypeStruct((_NP, _DW), jnp.float32),
        ],
    )(degp, x_p, c0, b0)


def _comb_body(h_ref, p_ref, inv_ref, q_ref, t_ref):
    q = 0.5 * (h_ref[...] + p_ref[0] + p_ref[1])
    q_ref[...] = q
    t_ref[...] = q * inv_ref[:, :1]


def _comb(h, p, inv16):
    return pl.pallas_call(
        _comb_body,
        grid=(_GRID,),
        in_specs=[
            pl.BlockSpec((_BR, _F), _row),
            pl.BlockSpec((_NCORE, _BR, _F), _prow),
            pl.BlockSpec((_BR, _DW), _row),
        ],
        out_specs=[
            pl.BlockSpec((_BR, _F), _row),
            pl.BlockSpec((_BR, _F), _row),
        ],
        out_shape=[
            jax.ShapeDtypeStruct((_NP, _F), jnp.float32),
            jax.ShapeDtypeStruct((_NP, _F), jnp.float32),
        ],
    )(h, p, inv16)


def _attn_body(q1_ref, q2_ref, q3_ref, q4_ref, s_ref,
               wcat_ref, ba_ref, b4_ref, c1_ref, b1_ref, wg_ref, bg_ref,
               u_ref, us_ref):
    f32 = jnp.float32
    q1 = q1_ref[...]
    q2 = q2_ref[...]
    q3 = q3_ref[...]
    q4 = q4_ref[...]
    wcat = wcat_ref[...]
    ba = ba_ref[...]
    b4 = b4_ref[...]

    def chan(base, take_abs):
        ch = jnp.dot(base, wcat, preferred_element_type=f32)
        if take_abs:
            ch = jnp.abs(ch)
        e = jnp.dot(ch, ba, preferred_element_type=f32)
        e = jnp.where(e >= 0, e, 0.2 * e)
        return ch, e

    chs = []
    es = []
    for base, take_abs in ((q1, False), (q2, False), (q3, False),
                           (q1 - q2, True), (q2 - q4, True)):
        ch, e = chan(base, take_abs)
        chs.append(ch)
        es.append(e)

    m = es[0]
    for e in es[1:]:
        m = jnp.maximum(m, e)
    ws = [jnp.exp(e - m) for e in es]
    z = ws[0] + ws[1] + ws[2] + ws[3] + ws[4]
    o = jnp.zeros_like(chs[0])
    for ch, w in zip(chs, ws):
        alpha = w / z
        o = o + ch * jnp.dot(alpha, b4, preferred_element_type=f32)

    h1 = o * c1_ref[...] + b1_ref[...]
    h1 = jnp.maximum(h1, 0.0)
    u = jnp.dot(h1, wg_ref[...], preferred_element_type=f32) + bg_ref[...]
    u_ref[...] = u
    us_ref[...] = u * s_ref[:, :1]


def _attn(q1, q2, q3, q4, s16, wcat, ba, b4, c1, b1, wg2, bg2):
    fw = _NH * _F
    return pl.pallas_call(
        _attn_body,
        grid=(_GRID,),
        in_specs=[
            pl.BlockSpec((_BR, _F), _row),
            pl.BlockSpec((_BR, _F), _row),
            pl.BlockSpec((_BR, _F), _row),
            pl.BlockSpec((_BR, _F), _row),
            pl.BlockSpec((_BR, _DW), _row),
            pl.BlockSpec((_F, fw), _fix),
            pl.BlockSpec((fw, _F), _fix),
            pl.BlockSpec((_F, fw), _fix),
            pl.BlockSpec((1, fw), _fix),
            pl.BlockSpec((1, fw), _fix),
            pl.BlockSpec((fw, _F), _fix),
            pl.BlockSpec((1, _F), _fix),
        ],
        out_specs=[
            pl.BlockSpec((_BR, _F), _row),
            pl.BlockSpec((_BR, _F), _row),
        ],
        out_shape=[
            jax.ShapeDtypeStruct((_NP, _F), jnp.float32),
            jax.ShapeDtypeStruct((_NP, _F), jnp.float32),
        ],
    )(q1, q2, q3, q4, s16, wcat, ba, b4, c1, b1, wg2, bg2)


def _final_body(u_ref, v_ref, s_ref, c3_ref, b3_ref, wfc_ref, bfc_ref,
                out_ref):
    agg = (v_ref[0] + v_ref[1]) * s_ref[:, :1]
    z = (_SM * agg + u_ref[...]) / (1.0 + _SM)
    z = z * c3_ref[...] + b3_ref[...]
    z = jnp.where(z >= 0, z, 0.01 * z)
    l = jnp.dot(z, wfc_ref[...], preferred_element_type=jnp.float32)
    l = l + bfc_ref[...]
    m = jnp.max(l, axis=1, keepdims=True)
    lse = jnp.log(jnp.sum(jnp.exp(l - m), axis=1, keepdims=True)) + m
    out_ref[...] = l - lse


def _final(u, v, s16, c3, b3, wfc_p, bfc_p):
    return pl.pallas_call(
        _final_body,
        grid=(_GRID,),
        in_specs=[
            pl.BlockSpec((_BR, _F), _row),
            pl.BlockSpec((_NCORE, _BR, _F), _prow),
            pl.BlockSpec((_BR, _DW), _row),
            pl.BlockSpec((1, _F), _fix),
            pl.BlockSpec((1, _F), _fix),
            pl.BlockSpec((_F, _F), _fix),
            pl.BlockSpec((1, _F), _fix),
        ],
        out_specs=pl.BlockSpec((_BR, _F), _row),
        out_shape=jax.ShapeDtypeStruct((_NP, _F), jnp.float32),
    )(u, v, s16, c3, b3, wfc_p, bfc_p)


def kernel(x, adj, W_att, a_att, g0, b0, g1, b1, g2, b2, g3, b3,
           Wg, bg, Wfc, bfc):
    f32 = jnp.float32
    fw = _NH * _F
    pad = _EP - _E
    padrows = _N + (jnp.arange(pad, dtype=jnp.int32) % (_NP - _N))
    src3 = jnp.concatenate([adj[0], padrows]).reshape(_NW, _CPT, _CHK)
    dst3 = jnp.concatenate([adj[1], padrows]).reshape(_NW, _CPT, _CHK)
    x_p = jnp.pad(x, ((0, _NP - _N), (0, 0)))
    zerosF = jnp.zeros((_NP, _F), f32)
    onesF = jnp.ones((_CHK, _F), f32)

    sc = 1.0 / jnp.sqrt(jnp.asarray(1.0 + _EPS, f32))
    c0 = (g0 * sc)[None]
    b0r = b0[None]
    wcat = jnp.transpose(W_att, (1, 0, 2)).reshape(_F, fw)
    ba = jnp.zeros((fw, _F), f32)
    for i in range(_NH):
        ba = ba.at[i * _F:(i + 1) * _F, i].set(a_att[i])
    b4 = jnp.zeros((_F, fw), f32)
    for i in range(_NH):
        b4 = b4.at[i, i * _F:(i + 1) * _F].set(1.0)
    c1 = jnp.tile(g1 * sc, _NH)[None]
    b1r = jnp.tile(b1, _NH)[None]
    wg2 = (g2 * sc)[:, None] * Wg
    bg2 = (b2 @ Wg + bg)[None]
    c3 = (g3 * sc)[None]
    b3r = b3[None]
    wfc_p = jnp.pad(Wfc, ((0, 0), (0, _F - _NCLS)))
    bfc_p = jnp.concatenate([bfc, jnp.full((_F - _NCLS,), -1e30, f32)])[None]

    degp = _deg(dst3, onesF, zerosF)
    xt, t0, inv16, s16 = _prep(degp, x_p, c0, b0r)
    p = _spmm(t0, src3, dst3, zerosF)
    q1, t1 = _comb(xt, p, inv16)
    p = _spmm(t1, src3, dst3, zerosF)
    q2, t2 = _comb(q1, p, inv16)
    p = _spmm(t2, src3, dst3, zerosF)
    q3, t3 = _comb(q2, p, inv16)
    p = _spmm(t3, src3, dst3, zerosF)
    q4, _ = _comb(q3, p, inv16)
    u, us = _attn(q1, q2, q3, q4, s16, wcat, ba, b4, c1, b1r, wg2, bg2)
    v = _spmm(us, src3, dst3, zerosF)
    lp = _final(u, v, s16, c3, b3r, wfc_p, bfc_p)
    return lp[:_N, :_NCLS]


# speedup vs baseline: 24.7628x; 1.6158x over previous
import functools

import jax
import jax.numpy as jnp
from jax import lax
from jax.experimental import pallas as pl
from jax.experimental.pallas import tpu as pltpu
from jax.experimental.pallas import tpu_sc as plsc

_N = 10000
_E = 320000
_F = 128
_NH = 4
_NCLS = 40
_EPS = 1e-5
_SM = 0.5

_NP = 10240
_NCORE = 2
_NSUB = 16
_NW = _NCORE * _NSUB
_CHK = 128
_CPT = 80
_DCHK = 64
_DCPT = 160
_EP = _NW * _CPT * _CHK
_RPT = _NP // _NSUB
_DW = 16

_BR = 256
_GRID = _NP // _BR


@functools.cache
def _make_spmm():
    mesh = plsc.VectorSubcoreMesh(
        core_axis_name="c", subcore_axis_name="s",
        num_cores=_NCORE, num_subcores=_NSUB)

    @functools.partial(
        pl.kernel,
        out_type=jax.ShapeDtypeStruct((_NCORE, _NP, _F), jnp.float32),
        mesh=mesh,
        scratch_types=[
            pltpu.VMEM((4, _CHK), jnp.int32),
            pltpu.VMEM((4, _CHK), jnp.int32),
            pltpu.VMEM((2, _CHK, _F), jnp.float32),
            pltpu.VMEM_SHARED((_NP, _F), jnp.float32),
            pltpu.SemaphoreType.DMA,
            pltpu.SemaphoreType.DMA,
            pltpu.SemaphoreType.DMA,
        ],
    )
    def spmm(x_hbm, src_hbm, dst_hbm, zeros_hbm, out_hbm,
             sidx, didx, rows, acc, isem, gsem, ssem):
        c = lax.axis_index("c")
        s = lax.axis_index("s")
        wid = s * _NCORE + c
        r0 = s * _RPT

        def load_pair(k, slot):
            pltpu.async_copy(src_hbm.at[wid, k], sidx.at[slot], isem)
            pltpu.async_copy(dst_hbm.at[wid, k], didx.at[slot], isem)

        def wait_pair(slot):
            pltpu.make_async_copy(src_hbm.at[wid, 0], sidx.at[slot],
                                  isem).wait()
            pltpu.make_async_copy(dst_hbm.at[wid, 0], didx.at[slot],
                                  isem).wait()

        pltpu.sync_copy(src_hbm.at[wid, 0], sidx.at[0])
        pltpu.sync_copy(dst_hbm.at[wid, 0], didx.at[0])
        load_pair(1, 1)
        load_pair(2, 2)
        pltpu.async_copy(x_hbm.at[sidx.at[0]], rows.at[0], gsem)
        pltpu.sync_copy(zeros_hbm.at[pl.ds(r0, _RPT)],
                        acc.at[pl.ds(r0, _RPT)])
        plsc.subcore_barrier()

        def body(k, carry):
            gslot = lax.rem(k, 2)
            islot = lax.rem(k, 4)
            nslot = lax.rem(k + 1, 4)

            @pl.when(k + 1 < _CPT)
            def _():
                wait_pair(nslot)

            pltpu.make_async_copy(x_hbm.at[sidx.at[islot]],
                                  rows.at[gslot], gsem).wait()

            @pl.when(k + 1 < _CPT)
            def _():
                pltpu.async_copy(x_hbm.at[sidx.at[nslot]],
                                 rows.at[1 - gslot], gsem)

            @pl.when(k + 3 < _CPT)
            def _():
                load_pair(k + 3, lax.rem(k + 3, 4))

            pltpu.async_copy(rows.at[gslot], acc.at[didx.at[islot]],
                             ssem, add=True)
            pltpu.make_async_copy(rows.at[gslot], acc.at[didx.at[islot]],
                                  ssem).wait()
            return carry

        lax.fori_loop(0, _CPT, body, 0)
        plsc.subcore_barrier()
        pltpu.sync_copy(acc.at[pl.ds(r0, _RPT)],
                        out_hbm.at[c, pl.ds(r0, _RPT)])

    return spmm


@functools.cache
def _make_deg():
    mesh = plsc.VectorSubcoreMesh(
        core_axis_name="c", subcore_axis_name="s",
        num_cores=_NCORE, num_subcores=_NSUB)

    @functools.partial(
        pl.kernel,
        out_type=jax.ShapeDtypeStruct((_NCORE, _NP, _F), jnp.float32),
        mesh=mesh,
        scratch_types=[
            pltpu.VMEM((4, _DCHK), jnp.int32),
            pltpu.VMEM((_DCHK, _F), jnp.float32),
            pltpu.VMEM_SHARED((_NP, _F), jnp.float32),
            pltpu.SemaphoreType.DMA,
            pltpu.SemaphoreType.DMA,
        ],
    )
    def deg(dst_hbm, ones_hbm, zeros_hbm, out_hbm, didx, ones_v, acc,
            isem, ssem):
        c = lax.axis_index("c")
        s = lax.axis_index("s")
        wid = s * _NCORE + c
        r0 = s * _RPT
        pltpu.sync_copy(dst_hbm.at[wid, 0], didx.at[0])
        pltpu.async_copy(dst_hbm.at[wid, 1], didx.at[1], isem)
        pltpu.async_copy(dst_hbm.at[wid, 2], didx.at[2], isem)
        pltpu.sync_copy(zeros_hbm.at[pl.ds(r0, _RPT)],
                        acc.at[pl.ds(r0, _RPT)])
        pltpu.sync_copy(ones_hbm, ones_v)
        plsc.subcore_barrier()

        def body(k, carry):
            islot = lax.rem(k, 4)
            pltpu.async_copy(ones_v, acc.at[didx.at[islot]], ssem, add=True)

            @pl.when(k > 0)
            def _():
                pltpu.make_async_copy(ones_v, acc.at[didx.at[islot]],
                                      ssem).wait()

            @pl.when(k + 3 < _DCPT)
            def _():
                pltpu.async_copy(dst_hbm.at[wid, k + 3],
                                 didx.at[lax.rem(k + 3, 4)], isem)

            @pl.when(k + 1 < _DCPT)
            def _():
                pltpu.make_async_copy(dst_hbm.at[wid, 0],
                                      didx.at[lax.rem(k + 1, 4)],
                                      isem).wait()

            return carry

        lax.fori_loop(0, _DCPT, body, 0)
        pltpu.make_async_copy(ones_v, acc.at[didx.at[0]], ssem).wait()
        plsc.subcore_barrier()
        pltpu.sync_copy(acc.at[pl.ds(r0, _RPT)],
                        out_hbm.at[c, pl.ds(r0, _RPT)])

    return deg


def _spmm(x_p, src3, dst3, zerosF):
    return _make_spmm()(x_p, src3, dst3, zerosF)


def _deg(dst3, onesD, zerosD):
    return _make_deg()(dst3, onesD, zerosD)


def _row(i):
    return (i, 0)


def _fix(i):
    return (0, 0)


def _prow(i):
    return (0, i, 0)


def _prep_body(degp_ref, x_ref, c0_ref, b0_ref,
               xt_ref, t0_ref, inv_ref, s_ref):
    i = pl.program_id(0)
    deg = degp_ref[0, :, :_DW] + degp_ref[1, :, :_DW]
    deg = jnp.maximum(deg, 1.0)
    rows = lax.broadcasted_iota(jnp.int32, (_BR, _DW), 0) + i * _BR
    valid = rows < _N
    inv = jnp.where(valid, 1.0 / deg, 0.0)
    sq = jnp.where(valid, lax.rsqrt(deg), 0.0)
    inv_ref[...] = inv
    s_ref[...] = sq
    xt = x_ref[...] * c0_ref[...] + b0_ref[...]
    xt_ref[...] = xt
    t0_ref[...] = xt * inv[:, :1]


def _prep(degp, x_p, c0, b0):
    return pl.pallas_call(
        _prep_body,
        grid=(_GRID,),
        in_specs=[
            pl.BlockSpec((_NCORE, _BR, _F), _prow),
            pl.BlockSpec((_BR, _F), _row),
            pl.BlockSpec((1, _F), _fix),
            pl.BlockSpec((1, _F), _fix),
        ],
        out_specs=[
            pl.BlockSpec((_BR, _F), _row),
            pl.BlockSpec((_BR, _F), _row),
            pl.BlockSpec((_BR, _DW), _row),
            pl.BlockSpec((_BR, _DW), _row),
        ],
        out_shape=[
            jax.ShapeDtypeStruct((_NP, _F), jnp.float32),
            jax.ShapeDtypeStruct((_NP, _F), jnp.float32),
            jax.ShapeDtypeStruct((_NP, _DW), jnp.float32),
            jax.ShapeDtypeStruct((_NP, _DW), jnp.float32),
        ],
    )(degp, x_p, c0, b0)


def _comb_body(h_ref, p_ref, inv_ref, q_ref, t_ref):
    q = 0.5 * (h_ref[...] + p_ref[0] + p_ref[1])
    q_ref[...] = q
    t_ref[...] = q * inv_ref[:, :1]


def _comb(h, p, inv16):
    return pl.pallas_call(
        _comb_body,
        grid=(_GRID,),
        in_specs=[
            pl.BlockSpec((_BR, _F), _row),
            pl.BlockSpec((_NCORE, _BR, _F), _prow),
            pl.BlockSpec((_BR, _DW), _row),
        ],
        out_specs=[
            pl.BlockSpec((_BR, _F), _row),
            pl.BlockSpec((_BR, _F), _row),
        ],
        out_shape=[
            jax.ShapeDtypeStruct((_NP, _F), jnp.float32),
            jax.ShapeDtypeStruct((_NP, _F), jnp.float32),
        ],
    )(h, p, inv16)


def _attn_body(q1_ref, q2_ref, q3_ref, q4_ref, s_ref,
               wcat_ref, ba_ref, b4_ref, c1_ref, b1_ref, wg_ref, bg_ref,
               u_ref, us_ref):
    f32 = jnp.float32
    q1 = q1_ref[...]
    q2 = q2_ref[...]
    q3 = q3_ref[...]
    q4 = q4_ref[...]
    wcat = wcat_ref[...]
    ba = ba_ref[...]
    b4 = b4_ref[...]

    def chan(base, take_abs):
        ch = jnp.dot(base, wcat, preferred_element_type=f32)
        if take_abs:
            ch = jnp.abs(ch)
        e = jnp.dot(ch, ba, preferred_element_type=f32)
        e = jnp.where(e >= 0, e, 0.2 * e)
        return ch, e

    chs = []
    es = []
    for base, take_abs in ((q1, False), (q2, False), (q3, False),
                           (q1 - q2, True), (q2 - q4, True)):
        ch, e = chan(base, take_abs)
        chs.append(ch)
        es.append(e)

    m = es[0]
    for e in es[1:]:
        m = jnp.maximum(m, e)
    ws = [jnp.exp(e - m) for e in es]
    z = ws[0] + ws[1] + ws[2] + ws[3] + ws[4]
    o = jnp.zeros_like(chs[0])
    for ch, w in zip(chs, ws):
        alpha = w / z
        o = o + ch * jnp.dot(alpha, b4, preferred_element_type=f32)

    h1 = o * c1_ref[...] + b1_ref[...]
    h1 = jnp.maximum(h1, 0.0)
    u = jnp.dot(h1, wg_ref[...], preferred_element_type=f32) + bg_ref[...]
    u_ref[...] = u
    us_ref[...] = u * s_ref[:, :1]


def _attn(q1, q2, q3, q4, s16, wcat, ba, b4, c1, b1, wg2, bg2):
    fw = _NH * _F
    return pl.pallas_call(
        _attn_body,
        grid=(_GRID,),
        in_specs=[
            pl.BlockSpec((_BR, _F), _row),
            pl.BlockSpec((_BR, _F), _row),
            pl.BlockSpec((_BR, _F), _row),
            pl.BlockSpec((_BR, _F), _row),
            pl.BlockSpec((_BR, _DW), _row),
            pl.BlockSpec((_F, fw), _fix),
            pl.BlockSpec((fw, _F), _fix),
            pl.BlockSpec((_F, fw), _fix),
            pl.BlockSpec((1, fw), _fix),
            pl.BlockSpec((1, fw), _fix),
            pl.BlockSpec((fw, _F), _fix),
            pl.BlockSpec((1, _F), _fix),
        ],
        out_specs=[
            pl.BlockSpec((_BR, _F), _row),
            pl.BlockSpec((_BR, _F), _row),
        ],
        out_shape=[
            jax.ShapeDtypeStruct((_NP, _F), jnp.float32),
            jax.ShapeDtypeStruct((_NP, _F), jnp.float32),
        ],
    )(q1, q2, q3, q4, s16, wcat, ba, b4, c1, b1, wg2, bg2)


def _final_body(u_ref, v_ref, s_ref, c3_ref, b3_ref, wfc_ref, bfc_ref,
                out_ref):
    agg = (v_ref[0] + v_ref[1]) * s_ref[:, :1]
    z = (_SM * agg + u_ref[...]) / (1.0 + _SM)
    z = z * c3_ref[...] + b3_ref[...]
    z = jnp.where(z >= 0, z, 0.01 * z)
    l = jnp.dot(z, wfc_ref[...], preferred_element_type=jnp.float32)
    l = l + bfc_ref[...]
    m = jnp.max(l, axis=1, keepdims=True)
    lse = jnp.log(jnp.sum(jnp.exp(l - m), axis=1, keepdims=True)) + m
    out_ref[...] = l - lse


def _final(u, v, s16, c3, b3, wfc_p, bfc_p):
    return pl.pallas_call(
        _final_body,
        grid=(_GRID,),
        in_specs=[
            pl.BlockSpec((_BR, _F), _row),
            pl.BlockSpec((_NCORE, _BR, _F), _prow),
            pl.BlockSpec((_BR, _DW), _row),
            pl.BlockSpec((1, _F), _fix),
            pl.BlockSpec((1, _F), _fix),
            pl.BlockSpec((_F, _F), _fix),
            pl.BlockSpec((1, _F), _fix),
        ],
        out_specs=pl.BlockSpec((_BR, _F), _row),
        out_shape=jax.ShapeDtypeStruct((_NP, _F), jnp.float32),
    )(u, v, s16, c3, b3, wfc_p, bfc_p)


def kernel(x, adj, W_att, a_att, g0, b0, g1, b1, g2, b2, g3, b3,
           Wg, bg, Wfc, bfc):
    f32 = jnp.float32
    fw = _NH * _F
    pad = _EP - _E
    padrows = _N + (jnp.arange(pad, dtype=jnp.int32) % (_NP - _N))
    srcflat = jnp.concatenate([adj[0], padrows])
    dstflat = jnp.concatenate([adj[1], padrows])
    src3 = srcflat.reshape(_NW, _CPT, _CHK)
    dst3 = dstflat.reshape(_NW, _CPT, _CHK)
    dstD = dstflat.reshape(_NW, _DCPT, _DCHK)
    x_p = jnp.pad(x, ((0, _NP - _N), (0, 0)))
    zerosF = jnp.zeros((_NP, _F), f32)
    onesF = jnp.ones((_DCHK, _F), f32)

    sc = 1.0 / jnp.sqrt(jnp.asarray(1.0 + _EPS, f32))
    c0 = (g0 * sc)[None]
    b0r = b0[None]
    wcat = jnp.transpose(W_att, (1, 0, 2)).reshape(_F, fw)
    ba = jnp.zeros((fw, _F), f32)
    for i in range(_NH):
        ba = ba.at[i * _F:(i + 1) * _F, i].set(a_att[i])
    b4 = jnp.zeros((_F, fw), f32)
    for i in range(_NH):
        b4 = b4.at[i, i * _F:(i + 1) * _F].set(1.0)
    c1 = jnp.tile(g1 * sc, _NH)[None]
    b1r = jnp.tile(b1, _NH)[None]
    wg2 = (g2 * sc)[:, None] * Wg
    bg2 = (b2 @ Wg + bg)[None]
    c3 = (g3 * sc)[None]
    b3r = b3[None]
    wfc_p = jnp.pad(Wfc, ((0, 0), (0, _F - _NCLS)))
    bfc_p = jnp.concatenate([bfc, jnp.full((_F - _NCLS,), -1e30, f32)])[None]

    degp = _deg(dstD, onesF, zerosF)
    xt, t0, inv16, s16 = _prep(degp, x_p, c0, b0r)
    p = _spmm(t0, src3, dst3, zerosF)
    q1, t1 = _comb(xt, p, inv16)
    p = _spmm(t1, src3, dst3, zerosF)
    q2, t2 = _comb(q1, p, inv16)
    p = _spmm(t2, src3, dst3, zerosF)
    q3, t3 = _comb(q2, p, inv16)
    p = _spmm(t3, src3, dst3, zerosF)
    q4, _ = _comb(q3, p, inv16)
    u, us = _attn(q1, q2, q3, q4, s16, wcat, ba, b4, c1, b1r, wg2, bg2)
    v = _spmm(us, src3, dst3, zerosF)
    lp = _final(u, v, s16, c3, b3r, wfc_p, bfc_p)
    return lp[:_N, :_NCLS]


# speedup vs baseline: 25.4004x; 1.0257x over previous
import functools

import jax
import jax.numpy as jnp
from jax import lax
from jax.experimental import pallas as pl
from jax.experimental.pallas import tpu as pltpu
from jax.experimental.pallas import tpu_sc as plsc

_N = 10000
_E = 320000
_F = 128
_NH = 4
_NCLS = 40
_EPS = 1e-5
_SM = 0.5

_NP = 10240
_NCORE = 2
_NSUB = 16
_NW = _NCORE * _NSUB
_CHK = 128
_CPT = 80
_DCHK = 64
_DCPT = 160
_EP = _NW * _CPT * _CHK
_RPT = _NP // _NSUB
_DW = 16

_BR = 256
_GRID = _NP // _BR


@functools.cache
def _make_spmm():
    mesh = plsc.VectorSubcoreMesh(
        core_axis_name="c", subcore_axis_name="s",
        num_cores=_NCORE, num_subcores=_NSUB)

    @functools.partial(
        pl.kernel,
        out_type=jax.ShapeDtypeStruct((_NCORE, _NP, _F), jnp.float32),
        mesh=mesh,
        scratch_types=[
            pltpu.VMEM((4, _CHK), jnp.int32),
            pltpu.VMEM((4, _CHK), jnp.int32),
            pltpu.VMEM((2, _CHK, _F), jnp.float32),
            pltpu.VMEM_SHARED((_NP, _F), jnp.float32),
            pltpu.SemaphoreType.DMA,
            pltpu.SemaphoreType.DMA,
            pltpu.SemaphoreType.DMA,
        ],
    )
    def spmm(x_hbm, src_hbm, dst_hbm, zeros_hbm, out_hbm,
             sidx, didx, rows, acc, isem, gsem, ssem):
        c = lax.axis_index("c")
        s = lax.axis_index("s")
        wid = s * _NCORE + c
        r0 = s * _RPT

        def load_pair(k, slot):
            pltpu.async_copy(src_hbm.at[wid, k], sidx.at[slot], isem)
            pltpu.async_copy(dst_hbm.at[wid, k], didx.at[slot], isem)

        def wait_pair(slot):
            pltpu.make_async_copy(src_hbm.at[wid, 0], sidx.at[slot],
                                  isem).wait()
            pltpu.make_async_copy(dst_hbm.at[wid, 0], didx.at[slot],
                                  isem).wait()

        pltpu.sync_copy(src_hbm.at[wid, 0], sidx.at[0])
        pltpu.sync_copy(dst_hbm.at[wid, 0], didx.at[0])
        load_pair(1, 1)
        load_pair(2, 2)
        pltpu.async_copy(x_hbm.at[sidx.at[0]], rows.at[0], gsem)
        pltpu.sync_copy(zeros_hbm.at[pl.ds(r0, _RPT)],
                        acc.at[pl.ds(r0, _RPT)])
        plsc.subcore_barrier()

        def body(k, carry):
            gslot = lax.rem(k, 2)
            islot = lax.rem(k, 4)
            nslot = lax.rem(k + 1, 4)

            @pl.when(k + 1 < _CPT)
            def _():
                wait_pair(nslot)

            pltpu.make_async_copy(x_hbm.at[sidx.at[islot]],
                                  rows.at[gslot], gsem).wait()

            @pl.when(k > 0)
            def _():
                pltpu.make_async_copy(rows.at[1 - gslot],
                                      acc.at[didx.at[islot]], ssem).wait()

            @pl.when(k + 1 < _CPT)
            def _():
                pltpu.async_copy(x_hbm.at[sidx.at[nslot]],
                                 rows.at[1 - gslot], gsem)

            @pl.when(k + 3 < _CPT)
            def _():
                load_pair(k + 3, lax.rem(k + 3, 4))

            pltpu.async_copy(rows.at[gslot], acc.at[didx.at[islot]],
                             ssem, add=True)
            return carry

        lax.fori_loop(0, _CPT, body, 0)
        pltpu.make_async_copy(rows.at[0], acc.at[didx.at[0]], ssem).wait()
        plsc.subcore_barrier()
        pltpu.sync_copy(acc.at[pl.ds(r0, _RPT)],
                        out_hbm.at[c, pl.ds(r0, _RPT)])

    return spmm


@functools.cache
def _make_deg(width=_F):
    mesh = plsc.VectorSubcoreMesh(
        core_axis_name="c", subcore_axis_name="s",
        num_cores=_NCORE, num_subcores=_NSUB)

    @functools.partial(
        pl.kernel,
        out_type=jax.ShapeDtypeStruct((_NCORE, _NP, width), jnp.float32),
        mesh=mesh,
        scratch_types=[
            pltpu.VMEM((4, _DCHK), jnp.int32),
            pltpu.VMEM((_DCHK, width), jnp.float32),
            pltpu.VMEM_SHARED((_NP, width), jnp.float32),
            pltpu.SemaphoreType.DMA,
            pltpu.SemaphoreType.DMA,
        ],
    )
    def deg(dst_hbm, ones_hbm, zeros_hbm, out_hbm, didx, ones_v, acc,
            isem, ssem):
        c = lax.axis_index("c")
        s = lax.axis_index("s")
        wid = s * _NCORE + c
        r0 = s * _RPT
        pltpu.sync_copy(dst_hbm.at[wid, 0], didx.at[0])
        pltpu.async_copy(dst_hbm.at[wid, 1], didx.at[1], isem)
        pltpu.async_copy(dst_hbm.at[wid, 2], didx.at[2], isem)
        pltpu.sync_copy(zeros_hbm.at[pl.ds(r0, _RPT)],
                        acc.at[pl.ds(r0, _RPT)])
        pltpu.sync_copy(ones_hbm, ones_v)
        plsc.subcore_barrier()

        def body(k, carry):
            islot = lax.rem(k, 4)
            pltpu.async_copy(ones_v, acc.at[didx.at[islot]], ssem, add=True)

            @pl.when(k > 0)
            def _():
                pltpu.make_async_copy(ones_v, acc.at[didx.at[islot]],
                                      ssem).wait()

            @pl.when(k + 3 < _DCPT)
            def _():
                pltpu.async_copy(dst_hbm.at[wid, k + 3],
                                 didx.at[lax.rem(k + 3, 4)], isem)

            @pl.when(k + 1 < _DCPT)
            def _():
                pltpu.make_async_copy(dst_hbm.at[wid, 0],
                                      didx.at[lax.rem(k + 1, 4)],
                                      isem).wait()

            return carry

        lax.fori_loop(0, _DCPT, body, 0)
        pltpu.make_async_copy(ones_v, acc.at[didx.at[0]], ssem).wait()
        plsc.subcore_barrier()
        pltpu.sync_copy(acc.at[pl.ds(r0, _RPT)],
                        out_hbm.at[c, pl.ds(r0, _RPT)])

    return deg


def _spmm(x_p, src3, dst3, zerosF):
    return _make_spmm()(x_p, src3, dst3, zerosF)


def _deg(dst3, onesD, zerosD, width=_F):
    return _make_deg(width)(dst3, onesD, zerosD)


def _row(i):
    return (i, 0)


def _fix(i):
    return (0, 0)


def _prow(i):
    return (0, i, 0)


def _prep_body(degp_ref, x_ref, c0_ref, b0_ref,
               xt_ref, t0_ref, inv_ref, s_ref):
    i = pl.program_id(0)
    deg = degp_ref[0, :, :_DW] + degp_ref[1, :, :_DW]
    deg = jnp.maximum(deg, 1.0)
    rows = lax.broadcasted_iota(jnp.int32, (_BR, _DW), 0) + i * _BR
    valid = rows < _N
    inv = jnp.where(valid, 1.0 / deg, 0.0)
    sq = jnp.where(valid, lax.rsqrt(deg), 0.0)
    inv_ref[...] = inv
    s_ref[...] = sq
    xt = x_ref[...] * c0_ref[...] + b0_ref[...]
    xt_ref[...] = xt
    t0_ref[...] = xt * inv[:, :1]


def _prep(degp, x_p, c0, b0):
    return pl.pallas_call(
        _prep_body,
        grid=(_GRID,),
        in_specs=[
            pl.BlockSpec((_NCORE, _BR, _F), _prow),
            pl.BlockSpec((_BR, _F), _row),
            pl.BlockSpec((1, _F), _fix),
            pl.BlockSpec((1, _F), _fix),
        ],
        out_specs=[
            pl.BlockSpec((_BR, _F), _row),
            pl.BlockSpec((_BR, _F), _row),
            pl.BlockSpec((_BR, _DW), _row),
            pl.BlockSpec((_BR, _DW), _row),
        ],
        out_shape=[
            jax.ShapeDtypeStruct((_NP, _F), jnp.float32),
            jax.ShapeDtypeStruct((_NP, _F), jnp.float32),
            jax.ShapeDtypeStruct((_NP, _DW), jnp.float32),
            jax.ShapeDtypeStruct((_NP, _DW), jnp.float32),
        ],
    )(degp, x_p, c0, b0)


def _comb_body(h_ref, p_ref, inv_ref, q_ref, t_ref):
    q = 0.5 * (h_ref[...] + p_ref[0] + p_ref[1])
    q_ref[...] = q
    t_ref[...] = q * inv_ref[:, :1]


def _comb(h, p, inv16):
    return pl.pallas_call(
        _comb_body,
        grid=(_GRID,),
        in_specs=[
            pl.BlockSpec((_BR, _F), _row),
            pl.BlockSpec((_NCORE, _BR, _F), _prow),
            pl.BlockSpec((_BR, _DW), _row),
        ],
        out_specs=[
            pl.BlockSpec((_BR, _F), _row),
            pl.BlockSpec((_BR, _F), _row),
        ],
        out_shape=[
            jax.ShapeDtypeStruct((_NP, _F), jnp.float32),
            jax.ShapeDtypeStruct((_NP, _F), jnp.float32),
        ],
    )(h, p, inv16)


def _attn_body(q1_ref, q2_ref, q3_ref, p4_ref, s_ref,
               wcat_ref, ba_ref, b4_ref, c1_ref, b1_ref, wg_ref, bg_ref,
               u_ref, us_ref):
    f32 = jnp.float32
    q1 = q1_ref[...]
    q2 = q2_ref[...]
    q3 = q3_ref[...]
    q4 = 0.5 * (q3 + p4_ref[0] + p4_ref[1])
    wcat = wcat_ref[...]
    ba = ba_ref[...]
    b4 = b4_ref[...]

    def chan(base, take_abs):
        ch = jnp.dot(base, wcat, preferred_element_type=f32)
        if take_abs:
            ch = jnp.abs(ch)
        e = jnp.dot(ch, ba, preferred_element_type=f32)
        e = jnp.where(e >= 0, e, 0.2 * e)
        return ch, e

    chs = []
    es = []
    for base, take_abs in ((q1, False), (q2, False), (q3, False),
                           (q1 - q2, True), (q2 - q4, True)):
        ch, e = chan(base, take_abs)
        chs.append(ch)
        es.append(e)

    m = es[0]
    for e in es[1:]:
        m = jnp.maximum(m, e)
    ws = [jnp.exp(e - m) for e in es]
    z = ws[0] + ws[1] + ws[2] + ws[3] + ws[4]
    o = jnp.zeros_like(chs[0])
    for ch, w in zip(chs, ws):
        alpha = w / z
        o = o + ch * jnp.dot(alpha, b4, preferred_element_type=f32)

    h1 = o * c1_ref[...] + b1_ref[...]
    h1 = jnp.maximum(h1, 0.0)
    u = jnp.dot(h1, wg_ref[...], preferred_element_type=f32) + bg_ref[...]
    u_ref[...] = u
    us_ref[...] = u * s_ref[:, :1]


def _attn(q1, q2, q3, p4, s16, wcat, ba, b4, c1, b1, wg2, bg2):
    fw = _NH * _F
    return pl.pallas_call(
        _attn_body,
        grid=(_GRID,),
        in_specs=[
            pl.BlockSpec((_BR, _F), _row),
            pl.BlockSpec((_BR, _F), _row),
            pl.BlockSpec((_BR, _F), _row),
            pl.BlockSpec((_NCORE, _BR, _F), _prow),
            pl.BlockSpec((_BR, _DW), _row),
            pl.BlockSpec((_F, fw), _fix),
            pl.BlockSpec((fw, _F), _fix),
            pl.BlockSpec((_F, fw), _fix),
            pl.BlockSpec((1, fw), _fix),
            pl.BlockSpec((1, fw), _fix),
            pl.BlockSpec((fw, _F), _fix),
            pl.BlockSpec((1, _F), _fix),
        ],
        out_specs=[
            pl.BlockSpec((_BR, _F), _row),
            pl.BlockSpec((_BR, _F), _row),
        ],
        out_shape=[
            jax.ShapeDtypeStruct((_NP, _F), jnp.float32),
            jax.ShapeDtypeStruct((_NP, _F), jnp.float32),
        ],
    )(q1, q2, q3, p4, s16, wcat, ba, b4, c1, b1, wg2, bg2)


def _final_body(u_ref, v_ref, s_ref, c3_ref, b3_ref, wfc_ref, bfc_ref,
                out_ref):
    agg = (v_ref[0] + v_ref[1]) * s_ref[:, :1]
    z = (_SM * agg + u_ref[...]) / (1.0 + _SM)
    z = z * c3_ref[...] + b3_ref[...]
    z = jnp.where(z >= 0, z, 0.01 * z)
    l = jnp.dot(z, wfc_ref[...], preferred_element_type=jnp.float32)
    l = l + bfc_ref[...]
    m = jnp.max(l, axis=1, keepdims=True)
    lse = jnp.log(jnp.sum(jnp.exp(l - m), axis=1, keepdims=True)) + m
    out_ref[...] = l - lse


def _final(u, v, s16, c3, b3, wfc_p, bfc_p):
    return pl.pallas_call(
        _final_body,
        grid=(_GRID,),
        in_specs=[
            pl.BlockSpec((_BR, _F), _row),
            pl.BlockSpec((_NCORE, _BR, _F), _prow),
            pl.BlockSpec((_BR, _DW), _row),
            pl.BlockSpec((1, _F), _fix),
            pl.BlockSpec((1, _F), _fix),
            pl.BlockSpec((_F, _F), _fix),
            pl.BlockSpec((1, _F), _fix),
        ],
        out_specs=pl.BlockSpec((_BR, _F), _row),
        out_shape=jax.ShapeDtypeStruct((_NP, _F), jnp.float32),
    )(u, v, s16, c3, b3, wfc_p, bfc_p)


def kernel(x, adj, W_att, a_att, g0, b0, g1, b1, g2, b2, g3, b3,
           Wg, bg, Wfc, bfc):
    f32 = jnp.float32
    fw = _NH * _F
    pad = _EP - _E
    padrows = _N + (jnp.arange(pad, dtype=jnp.int32) % (_NP - _N))
    srcflat = jnp.concatenate([adj[0], padrows])
    dstflat = jnp.concatenate([adj[1], padrows])
    src3 = srcflat.reshape(_NW, _CPT, _CHK)
    dst3 = dstflat.reshape(_NW, _CPT, _CHK)
    dstD = dstflat.reshape(_NW, _DCPT, _DCHK)
    x_p = jnp.pad(x, ((0, _NP - _N), (0, 0)))
    zerosF = jnp.zeros((_NP, _F), f32)
    onesF = jnp.ones((_DCHK, _F), f32)

    sc = 1.0 / jnp.sqrt(jnp.asarray(1.0 + _EPS, f32))
    c0 = (g0 * sc)[None]
    b0r = b0[None]
    wcat = jnp.transpose(W_att, (1, 0, 2)).reshape(_F, fw)
    ba = jnp.zeros((fw, _F), f32)
    for i in range(_NH):
        ba = ba.at[i * _F:(i + 1) * _F, i].set(a_att[i])
    b4 = jnp.zeros((_F, fw), f32)
    for i in range(_NH):
        b4 = b4.at[i, i * _F:(i + 1) * _F].set(1.0)
    c1 = jnp.tile(g1 * sc, _NH)[None]
    b1r = jnp.tile(b1, _NH)[None]
    wg2 = (g2 * sc)[:, None] * Wg
    bg2 = (b2 @ Wg + bg)[None]
    c3 = (g3 * sc)[None]
    b3r = b3[None]
    wfc_p = jnp.pad(Wfc, ((0, 0), (0, _F - _NCLS)))
    bfc_p = jnp.concatenate([bfc, jnp.full((_F - _NCLS,), -1e30, f32)])[None]

    degp = _deg(dstD, onesF, zerosF)
    xt, t0, inv16, s16 = _prep(degp, x_p, c0, b0r)
    p = _spmm(t0, src3, dst3, zerosF)
    q1, t1 = _comb(xt, p, inv16)
    p = _spmm(t1, src3, dst3, zerosF)
    q2, t2 = _comb(q1, p, inv16)
    p = _spmm(t2, src3, dst3, zerosF)
    q3, t3 = _comb(q2, p, inv16)
    p4 = _spmm(t3, src3, dst3, zerosF)
    u, us = _attn(q1, q2, q3, p4, s16, wcat, ba, b4, c1, b1r, wg2, bg2)
    v = _spmm(us, src3, dst3, zerosF)
    lp = _final(u, v, s16, c3, b3r, wfc_p, bfc_p)
    return lp[:_N, :_NCLS]


# speedup vs baseline: 30.5973x; 1.2046x over previous
import functools

import jax
import jax.numpy as jnp
from jax import lax
from jax.experimental import pallas as pl
from jax.experimental.pallas import tpu as pltpu
from jax.experimental.pallas import tpu_sc as plsc

_N = 10000
_E = 320000
_F = 128
_NH = 4
_NCLS = 40
_EPS = 1e-5
_SM = 0.5

_NP = 10240
_NCORE = 2
_NSUB = 16
_NW = _NCORE * _NSUB
_CHK = 96
_CPT = 106
_DCHK = 64
_DCPT = 159
_EP = _NW * _CPT * _CHK
_RPT = _NP // _NSUB
_DW = 16

_BR = 256
_GRID = _NP // _BR


@functools.cache
def _make_spmm():
    mesh = plsc.VectorSubcoreMesh(
        core_axis_name="c", subcore_axis_name="s",
        num_cores=_NCORE, num_subcores=_NSUB)

    @functools.partial(
        pl.kernel,
        out_type=jax.ShapeDtypeStruct((_NCORE, _NP, _F), jnp.float32),
        mesh=mesh,
        scratch_types=[
            pltpu.VMEM((8, _CHK), jnp.int32),
            pltpu.VMEM((8, _CHK), jnp.int32),
            pltpu.VMEM((3, _CHK, _F), jnp.float32),
            pltpu.VMEM_SHARED((_NP, _F), jnp.float32),
            pltpu.SemaphoreType.DMA,
            pltpu.SemaphoreType.DMA,
            pltpu.SemaphoreType.DMA,
        ],
    )
    def spmm(x_hbm, src_hbm, dst_hbm, zeros_hbm, out_hbm,
             sidx, didx, rows, acc, isem, gsem, ssem):
        c = lax.axis_index("c")
        s = lax.axis_index("s")
        wid = s * _NCORE + c
        r0 = s * _RPT

        def load_pair(k, slot):
            pltpu.async_copy(src_hbm.at[wid, k], sidx.at[slot], isem)
            pltpu.async_copy(dst_hbm.at[wid, k], didx.at[slot], isem)

        def wait_pair(slot):
            pltpu.make_async_copy(src_hbm.at[wid, 0], sidx.at[slot],
                                  isem).wait()
            pltpu.make_async_copy(dst_hbm.at[wid, 0], didx.at[slot],
                                  isem).wait()

        def gather(k, islot, gslot):
            pltpu.async_copy(x_hbm.at[sidx.at[islot]], rows.at[gslot], gsem)

        pltpu.sync_copy(src_hbm.at[wid, 0], sidx.at[0])
        pltpu.sync_copy(dst_hbm.at[wid, 0], didx.at[0])
        load_pair(1, 1)
        load_pair(2, 2)
        load_pair(3, 3)
        gather(0, 0, 0)
        wait_pair(1)
        gather(1, 1, 1)
        pltpu.sync_copy(zeros_hbm.at[pl.ds(r0, _RPT)],
                        acc.at[pl.ds(r0, _RPT)])
        plsc.subcore_barrier()

        def body(k, carry):
            gslot = lax.rem(k, 3)
            islot = lax.rem(k, 8)

            @pl.when(k + 2 < _CPT)
            def _():
                wait_pair(lax.rem(k + 2, 8))

            pltpu.make_async_copy(x_hbm.at[sidx.at[islot]],
                                  rows.at[gslot], gsem).wait()

            @pl.when(k > 0)
            def _():
                pltpu.make_async_copy(rows.at[gslot],
                                      acc.at[didx.at[islot]], ssem).wait()

            @pl.when(k + 2 < _CPT)
            def _():
                gather(k + 2, lax.rem(k + 2, 8), lax.rem(k + 2, 3))

            @pl.when(k + 4 < _CPT)
            def _():
                load_pair(k + 4, lax.rem(k + 4, 8))

            pltpu.async_copy(rows.at[gslot], acc.at[didx.at[islot]],
                             ssem, add=True)
            return carry

        lax.fori_loop(0, _CPT, body, 0)
        pltpu.make_async_copy(rows.at[0], acc.at[didx.at[0]], ssem).wait()
        plsc.subcore_barrier()
        pltpu.sync_copy(acc.at[pl.ds(r0, _RPT)],
                        out_hbm.at[c, pl.ds(r0, _RPT)])

    return spmm


@functools.cache
def _make_deg(width=_F):
    mesh = plsc.VectorSubcoreMesh(
        core_axis_name="c", subcore_axis_name="s",
        num_cores=_NCORE, num_subcores=_NSUB)

    @functools.partial(
        pl.kernel,
        out_type=jax.ShapeDtypeStruct((_NCORE, _NP, width), jnp.float32),
        mesh=mesh,
        scratch_types=[
            pltpu.VMEM((8, _DCHK), jnp.int32),
            pltpu.VMEM((_DCHK, width), jnp.float32),
            pltpu.VMEM_SHARED((_NP, width), jnp.float32),
            pltpu.SemaphoreType.DMA,
            pltpu.SemaphoreType.DMA,
        ],
    )
    def deg(dst_hbm, ones_hbm, zeros_hbm, out_hbm, didx, ones_v, acc,
            isem, ssem):
        c = lax.axis_index("c")
        s = lax.axis_index("s")
        wid = s * _NCORE + c
        r0 = s * _RPT
        pltpu.sync_copy(dst_hbm.at[wid, 0], didx.at[0])
        for j in (1, 2, 3):
            pltpu.async_copy(dst_hbm.at[wid, j], didx.at[j], isem)
        pltpu.sync_copy(zeros_hbm.at[pl.ds(r0, _RPT)],
                        acc.at[pl.ds(r0, _RPT)])
        pltpu.sync_copy(ones_hbm, ones_v)
        pltpu.make_async_copy(dst_hbm.at[wid, 0], didx.at[1], isem).wait()
        plsc.subcore_barrier()

        def body(k, carry):
            islot = lax.rem(k, 8)
            pltpu.async_copy(ones_v, acc.at[didx.at[islot]], ssem, add=True)

            @pl.when(k >= 3)
            def _():
                pltpu.make_async_copy(ones_v, acc.at[didx.at[islot]],
                                      ssem).wait()

            @pl.when(k + 4 < _DCPT)
            def _():
                pltpu.async_copy(dst_hbm.at[wid, k + 4],
                                 didx.at[lax.rem(k + 4, 8)], isem)

            @pl.when(k + 2 < _DCPT)
            def _():
                pltpu.make_async_copy(dst_hbm.at[wid, 0],
                                      didx.at[lax.rem(k + 2, 8)],
                                      isem).wait()

            return carry

        lax.fori_loop(0, _DCPT, body, 0)
        for _ in range(3):
            pltpu.make_async_copy(ones_v, acc.at[didx.at[0]], ssem).wait()
        plsc.subcore_barrier()
        pltpu.sync_copy(acc.at[pl.ds(r0, _RPT)],
                        out_hbm.at[c, pl.ds(r0, _RPT)])

    return deg


def _spmm(x_p, src3, dst3, zerosF):
    return _make_spmm()(x_p, src3, dst3, zerosF)


def _deg(dst3, onesD, zerosD, width=_F):
    return _make_deg(width)(dst3, onesD, zerosD)


def _row(i):
    return (i, 0)


def _fix(i):
    return (0, 0)


def _prow(i):
    return (0, i, 0)


def _prep_body(degp_ref, x_ref, c0_ref, b0_ref,
               xt_ref, t0_ref, inv_ref, s_ref):
    i = pl.program_id(0)
    deg = degp_ref[0, :, :_DW] + degp_ref[1, :, :_DW]
    deg = jnp.maximum(deg, 1.0)
    rows = lax.broadcasted_iota(jnp.int32, (_BR, _DW), 0) + i * _BR
    valid = rows < _N
    inv = jnp.where(valid, 1.0 / deg, 0.0)
    sq = jnp.where(valid, lax.rsqrt(deg), 0.0)
    inv_ref[...] = inv
    s_ref[...] = sq
    xt = x_ref[...] * c0_ref[...] + b0_ref[...]
    xt_ref[...] = xt
    t0_ref[...] = xt * inv[:, :1]


def _prep(degp, x_p, c0, b0):
    return pl.pallas_call(
        _prep_body,
        grid=(_GRID,),
        in_specs=[
            pl.BlockSpec((_NCORE, _BR, _F), _prow),
            pl.BlockSpec((_BR, _F), _row),
            pl.BlockSpec((1, _F), _fix),
            pl.BlockSpec((1, _F), _fix),
        ],
        out_specs=[
            pl.BlockSpec((_BR, _F), _row),
            pl.BlockSpec((_BR, _F), _row),
            pl.BlockSpec((_BR, _DW), _row),
            pl.BlockSpec((_BR, _DW), _row),
        ],
        out_shape=[
            jax.ShapeDtypeStruct((_NP, _F), jnp.float32),
            jax.ShapeDtypeStruct((_NP, _F), jnp.float32),
            jax.ShapeDtypeStruct((_NP, _DW), jnp.float32),
            jax.ShapeDtypeStruct((_NP, _DW), jnp.float32),
        ],
    )(degp, x_p, c0, b0)


def _comb_body(h_ref, p_ref, inv_ref, q_ref, t_ref):
    q = 0.5 * (h_ref[...] + p_ref[0] + p_ref[1])
    q_ref[...] = q
    t_ref[...] = q * inv_ref[:, :1]


def _comb(h, p, inv16):
    return pl.pallas_call(
        _comb_body,
        grid=(_GRID,),
        in_specs=[
            pl.BlockSpec((_BR, _F), _row),
            pl.BlockSpec((_NCORE, _BR, _F), _prow),
            pl.BlockSpec((_BR, _DW), _row),
        ],
        out_specs=[
            pl.BlockSpec((_BR, _F), _row),
            pl.BlockSpec((_BR, _F), _row),
        ],
        out_shape=[
            jax.ShapeDtypeStruct((_NP, _F), jnp.float32),
            jax.ShapeDtypeStruct((_NP, _F), jnp.float32),
        ],
    )(h, p, inv16)


def _attn_body(q1_ref, q2_ref, q3_ref, p4_ref, s_ref,
               wcat_ref, ba_ref, b4_ref, c1_ref, b1_ref, wg_ref, bg_ref,
               u_ref, us_ref):
    f32 = jnp.float32
    q1 = q1_ref[...]
    q2 = q2_ref[...]
    q3 = q3_ref[...]
    q4 = 0.5 * (q3 + p4_ref[0] + p4_ref[1])
    wcat = wcat_ref[...]
    ba = ba_ref[...]
    b4 = b4_ref[...]

    def chan(base, take_abs):
        ch = jnp.dot(base, wcat, preferred_element_type=f32)
        if take_abs:
            ch = jnp.abs(ch)
        e = jnp.dot(ch, ba, preferred_element_type=f32)
        e = jnp.where(e >= 0, e, 0.2 * e)
        return ch, e

    chs = []
    es = []
    for base, take_abs in ((q1, False), (q2, False), (q3, False),
                           (q1 - q2, True), (q2 - q4, True)):
        ch, e = chan(base, take_abs)
        chs.append(ch)
        es.append(e)

    m = es[0]
    for e in es[1:]:
        m = jnp.maximum(m, e)
    ws = [jnp.exp(e - m) for e in es]
    z = ws[0] + ws[1] + ws[2] + ws[3] + ws[4]
    o = jnp.zeros_like(chs[0])
    for ch, w in zip(chs, ws):
        alpha = w / z
        o = o + ch * jnp.dot(alpha, b4, preferred_element_type=f32)

    h1 = o * c1_ref[...] + b1_ref[...]
    h1 = jnp.maximum(h1, 0.0)
    u = jnp.dot(h1, wg_ref[...], preferred_element_type=f32) + bg_ref[...]
    u_ref[...] = u
    us_ref[...] = u * s_ref[:, :1]


def _attn(q1, q2, q3, p4, s16, wcat, ba, b4, c1, b1, wg2, bg2):
    fw = _NH * _F
    return pl.pallas_call(
        _attn_body,
        grid=(_GRID,),
        in_specs=[
            pl.BlockSpec((_BR, _F), _row),
            pl.BlockSpec((_BR, _F), _row),
            pl.BlockSpec((_BR, _F), _row),
            pl.BlockSpec((_NCORE, _BR, _F), _prow),
            pl.BlockSpec((_BR, _DW), _row),
            pl.BlockSpec((_F, fw), _fix),
            pl.BlockSpec((fw, _F), _fix),
            pl.BlockSpec((_F, fw), _fix),
            pl.BlockSpec((1, fw), _fix),
            pl.BlockSpec((1, fw), _fix),
            pl.BlockSpec((fw, _F), _fix),
            pl.BlockSpec((1, _F), _fix),
        ],
        out_specs=[
            pl.BlockSpec((_BR, _F), _row),
            pl.BlockSpec((_BR, _F), _row),
        ],
        out_shape=[
            jax.ShapeDtypeStruct((_NP, _F), jnp.float32),
            jax.ShapeDtypeStruct((_NP, _F), jnp.float32),
        ],
    )(q1, q2, q3, p4, s16, wcat, ba, b4, c1, b1, wg2, bg2)


def _final_body(u_ref, v_ref, s_ref, c3_ref, b3_ref, wfc_ref, bfc_ref,
                out_ref):
    agg = (v_ref[0] + v_ref[1]) * s_ref[:, :1]
    z = (_SM * agg + u_ref[...]) / (1.0 + _SM)
    z = z * c3_ref[...] + b3_ref[...]
    z = jnp.where(z >= 0, z, 0.01 * z)
    l = jnp.dot(z, wfc_ref[...], preferred_element_type=jnp.float32)
    l = l + bfc_ref[...]
    m = jnp.max(l, axis=1, keepdims=True)
    lse = jnp.log(jnp.sum(jnp.exp(l - m), axis=1, keepdims=True)) + m
    out_ref[...] = l - lse


def _final(u, v, s16, c3, b3, wfc_p, bfc_p):
    return pl.pallas_call(
        _final_body,
        grid=(_GRID,),
        in_specs=[
            pl.BlockSpec((_BR, _F), _row),
            pl.BlockSpec((_NCORE, _BR, _F), _prow),
            pl.BlockSpec((_BR, _DW), _row),
            pl.BlockSpec((1, _F), _fix),
            pl.BlockSpec((1, _F), _fix),
            pl.BlockSpec((_F, _F), _fix),
            pl.BlockSpec((1, _F), _fix),
        ],
        out_specs=pl.BlockSpec((_BR, _F), _row),
        out_shape=jax.ShapeDtypeStruct((_NP, _F), jnp.float32),
    )(u, v, s16, c3, b3, wfc_p, bfc_p)


def kernel(x, adj, W_att, a_att, g0, b0, g1, b1, g2, b2, g3, b3,
           Wg, bg, Wfc, bfc):
    f32 = jnp.float32
    fw = _NH * _F
    pad = _EP - _E
    padrows = _N + (jnp.arange(pad, dtype=jnp.int32) % (_NP - _N))
    srcflat = jnp.concatenate([adj[0], padrows])
    dstflat = jnp.concatenate([adj[1], padrows])
    src3 = srcflat.reshape(_NW, _CPT, _CHK)
    dst3 = dstflat.reshape(_NW, _CPT, _CHK)
    dstD = dstflat.reshape(_NW, _DCPT, _DCHK)
    x_p = jnp.pad(x, ((0, _NP - _N), (0, 0)))
    zerosF = jnp.zeros((_NP, _F), f32)
    onesF = jnp.ones((_DCHK, _F), f32)

    sc = 1.0 / jnp.sqrt(jnp.asarray(1.0 + _EPS, f32))
    c0 = (g0 * sc)[None]
    b0r = b0[None]
    wcat = jnp.transpose(W_att, (1, 0, 2)).reshape(_F, fw)
    ba = jnp.zeros((fw, _F), f32)
    for i in range(_NH):
        ba = ba.at[i * _F:(i + 1) * _F, i].set(a_att[i])
    b4 = jnp.zeros((_F, fw), f32)
    for i in range(_NH):
        b4 = b4.at[i, i * _F:(i + 1) * _F].set(1.0)
    c1 = jnp.tile(g1 * sc, _NH)[None]
    b1r = jnp.tile(b1, _NH)[None]
    wg2 = (g2 * sc)[:, None] * Wg
    bg2 = (b2 @ Wg + bg)[None]
    c3 = (g3 * sc)[None]
    b3r = b3[None]
    wfc_p = jnp.pad(Wfc, ((0, 0), (0, _F - _NCLS)))
    bfc_p = jnp.concatenate([bfc, jnp.full((_F - _NCLS,), -1e30, f32)])[None]

    degp = _deg(dstD, onesF, zerosF)
    xt, t0, inv16, s16 = _prep(degp, x_p, c0, b0r)
    p = _spmm(t0, src3, dst3, zerosF)
    q1, t1 = _comb(xt, p, inv16)
    p = _spmm(t1, src3, dst3, zerosF)
    q2, t2 = _comb(q1, p, inv16)
    p = _spmm(t2, src3, dst3, zerosF)
    q3, t3 = _comb(q2, p, inv16)
    p4 = _spmm(t3, src3, dst3, zerosF)
    u, us = _attn(q1, q2, q3, p4, s16, wcat, ba, b4, c1, b1r, wg2, bg2)
    v = _spmm(us, src3, dst3, zerosF)
    lp = _final(u, v, s16, c3, b3r, wfc_p, bfc_p)
    return lp[:_N, :_NCLS]
